# Initial kernel scaffold; baseline (speedup 1.0000x reference)
import jax
import jax.numpy as jnp
from jax.experimental import pallas as pl


def kernel(batch):
    raise NotImplementedError("write your pallas kernel here")


# speedup vs baseline: 164.7375x; 164.7375x over previous
import functools

import jax
import jax.numpy as jnp
from jax import lax
from jax.experimental import pallas as pl
from jax.experimental.pallas import tpu as pltpu
from jax.experimental.pallas import tpu_sc as plsc

BINS0, BINS1 = 256, 256
CH = 4
ROWW = BINS1 * CH
INV_D = 128.0
NC, NS, L = 2, 16, 16
NW = NC * NS
HB = 56
_LOS = list(range(0, BINS0, HB))
_HS = [min(HB, BINS0 - lo) for lo in _LOS]
NPASS = len(_LOS)


def _sc_body(n_points, spw, bt_hbm, out_hbm,
             xs_v, ys_v, grid_a, grid_b, sem_a, sem_b):
    nvec = n_points // L
    wid = lax.axis_index("s") * NC + lax.axis_index("c")
    onesf = jnp.ones((L,), jnp.float32)
    zerof = jnp.zeros((L,), jnp.float32)
    grids = [grid_a, grid_b]
    sems = [sem_a, sem_b]

    def zrow(i, _):
        r = i // (ROWW // L)
        c = (i % (ROWW // L)) * L
        grid_a[r, pl.ds(c, L)] = zerof
        grid_b[r, pl.ds(c, L)] = zerof
        return 0
    lax.fori_loop(0, HB * (ROWW // L), zrow, 0)

    def pass_scatter(grid_v, lo, hi, x_is_one):
        def it(i, _):
            base = i * L
            xv = xs_v[pl.ds(base, L)]
            yv = ys_v[pl.ds(base, L)]
            ixv = (xv * INV_D + 0.5).astype(jnp.int32)
            iyv = (yv * INV_D + 0.5).astype(jnp.int32)
            m = jnp.logical_and(ixv >= lo, ixv < hi)
            r = jnp.where(m, ixv - lo, 0)
            c = iyv * CH
            if x_is_one:
                plsc.store_scatter(grid_v, [r, c], onesf, mask=m)
                plsc.store_scatter(grid_v, [r, c + 1], onesf, mask=m)
                plsc.store_scatter(grid_v, [r, c + 2], xv, mask=m)
                plsc.store_scatter(grid_v, [r, c + 3], yv, mask=m)
            else:
                plsc.store_scatter(grid_v, [r, c], zerof, mask=m)
                plsc.store_scatter(grid_v, [r, c + 1], zerof, mask=m)
                plsc.store_scatter(grid_v, [r, c + 2], zerof, mask=m)
                plsc.store_scatter(grid_v, [r, c + 3], zerof, mask=m)
            return 0
        lax.fori_loop(0, nvec, it, 0)

    for s in range(spw):
        b = wid * spw + s
        pltpu.sync_copy(bt_hbm.at[b, 0], xs_v)
        pltpu.sync_copy(bt_hbm.at[b, 1], ys_v)

        def minmax(i, carry):
            mn, mx = carry
            ix = (xs_v[pl.ds(i * L, L)] * INV_D + 0.5).astype(jnp.int32)
            return jnp.minimum(mn, ix), jnp.maximum(mx, ix)

        big = jnp.full((L,), 2**30, jnp.int32)
        mn_v, mx_v = lax.fori_loop(0, nvec, minmax, (big, -big))
        mn = jnp.min(mn_v)
        mx = jnp.max(mx_v)

        has = [jnp.logical_and(mx >= lo, mn < lo + h)
               for lo, h in zip(_LOS, _HS)]

        for p in range(NPASS):
            k = p % 2
            lo, h = _LOS[p], _HS[p]
            if p >= 2:
                pltpu.make_async_copy(
                    grids[k].at[pl.ds(0, _HS[p - 2])],
                    out_hbm.at[b, pl.ds(_LOS[p - 2], _HS[p - 2])],
                    sems[k]).wait()

                @pl.when(has[p - 2])
                def _cleanup():
                    pass_scatter(grids[k], _LOS[p - 2],
                                 _LOS[p - 2] + _HS[p - 2], False)

            @pl.when(has[p])
            def _scatter():
                pass_scatter(grids[k], lo, lo + h, True)

            pltpu.async_copy(
                grids[k].at[pl.ds(0, h)],
                out_hbm.at[b, pl.ds(lo, h)],
                sems[k])

        for p in (NPASS - 2, NPASS - 1):
            k = p % 2
            pltpu.make_async_copy(
                grids[k].at[pl.ds(0, _HS[p])],
                out_hbm.at[b, pl.ds(_LOS[p], _HS[p])],
                sems[k]).wait()
            if s != spw - 1:
                @pl.when(has[p])
                def _cleanup_tail():
                    pass_scatter(grids[k], _LOS[p], _LOS[p] + _HS[p], False)


@functools.lru_cache(maxsize=None)
def _build(batch_size, n_points):
    spw = batch_size // NW
    mesh = plsc.VectorSubcoreMesh(
        core_axis_name="c", subcore_axis_name="s",
        num_cores=NC, num_subcores=NS)
    return pl.kernel(
        functools.partial(_sc_body, n_points, spw),
        out_type=jax.ShapeDtypeStruct((batch_size, BINS0, ROWW), jnp.float32),
        mesh=mesh,
        compiler_params=pltpu.CompilerParams(needs_layout_passes=False),
        scratch_types=[
            pltpu.VMEM((n_points,), jnp.float32),
            pltpu.VMEM((n_points,), jnp.float32),
            pltpu.VMEM((HB, ROWW), jnp.float32),
            pltpu.VMEM((HB, ROWW), jnp.float32),
            pltpu.SemaphoreType.DMA,
            pltpu.SemaphoreType.DMA,
        ],
    )


def kernel(batch):
    batch_size, n_points, _ = batch.shape
    bt = jnp.transpose(batch, (0, 2, 1))
    out = _build(batch_size, n_points)(bt)
    return out.reshape(batch_size, BINS0, BINS1, CH)
